# baseline (speedup 1.0000x reference)
import functools

import jax
import jax.numpy as jnp
from jax import lax
from jax.experimental import pallas as pl
from jax.experimental.pallas import tpu as pltpu
from jax.experimental.pallas import tpu_sc as plsc

VOCAB = 100000
D = 16
F = 26
B = 4096
L = 50
S = F + 1

_INFO = plsc.get_sparse_core_info()
NW = _INFO.num_cores * _INFO.num_subcores
BPW = B // NW
ROWS_PER_GATHER = 128
N_SP_CHUNK = BPW * S // ROWS_PER_GATHER
SEQ_SUB = 2
BPC = BPW // SEQ_SUB
SEQ_ROWS = BPC * L
N_SQ_CHUNK = SEQ_ROWS // ROWS_PER_GATHER
SP_STRIDE = 32
SQ_STRIDE = 56
INV_L = float(1.0 / L)


def _body(idx27_hbm, seqidx_hbm, sp_flat_hbm, seq_tab_hbm, out_hbm,
          idx_sp, idx_sq, out_blk, seq_rows, semS, semQ):
    wid = lax.axis_index("s") * _INFO.num_cores + lax.axis_index("c")

    pltpu.sync_copy(idx27_hbm.at[pl.ds(wid * SP_STRIDE, SP_STRIDE)], idx_sp)
    pltpu.sync_copy(seqidx_hbm.at[pl.ds(wid * SQ_STRIDE, SQ_STRIDE)], idx_sq)

    sp_cps = []
    for j in range(N_SP_CHUNK):
        sp_cps.append(pltpu.async_copy(
            sp_flat_hbm.at[idx_sp.at[j]],
            out_blk.at[pl.ds(j * ROWS_PER_GATHER, ROWS_PER_GATHER)],
            semS))

    for c in range(SEQ_SUB):
        sq_cps = []
        for j in range(N_SQ_CHUNK):
            sq_cps.append(pltpu.async_copy(
                seq_tab_hbm.at[idx_sq.at[c * N_SQ_CHUNK + j]],
                seq_rows.at[pl.ds(j * ROWS_PER_GATHER, ROWS_PER_GATHER)],
                semQ))
        for cp in sq_cps:
            cp.wait()
        if c == 0:
            for cp in sp_cps:
                cp.wait()

        def pool_one(bb, _, c=c):
            r0 = bb * L
            acc = seq_rows[r0, :]
            for l in range(1, L):
                acc = acc + seq_rows[r0 + l, :]
            out_blk[(c * BPC + bb) * S + F, :] = acc * INV_L
            return _

        lax.fori_loop(0, BPC, pool_one, None)

    pltpu.sync_copy(out_blk, out_hbm.at[pl.ds(wid * BPW * S, BPW * S)])


@functools.partial(jax.jit, static_argnames=())
def kernel(sparse_idx, seq_idx, sparse_tables, seq_table):
    offs = (jnp.arange(F, dtype=jnp.int32) * VOCAB)[None, :]
    idx27 = jnp.concatenate(
        [sparse_idx + offs, jnp.zeros((B, 1), jnp.int32)], axis=1)
    idx27_3d = idx27.reshape(NW, N_SP_CHUNK, 128)
    idx27_2d = jnp.pad(
        idx27_3d, ((0, 0), (0, SP_STRIDE - N_SP_CHUNK), (0, 0))
    ).reshape(NW * SP_STRIDE, 128)
    seq_3d = seq_idx.reshape(NW, BPW * L // 128, 128)
    seq_2d = jnp.pad(
        seq_3d, ((0, 0), (0, SQ_STRIDE - BPW * L // 128), (0, 0))
    ).reshape(NW * SQ_STRIDE, 128)
    sp_flat = sparse_tables.reshape(F * VOCAB, D)

    mesh = plsc.VectorSubcoreMesh(core_axis_name="c", subcore_axis_name="s")
    run = pl.kernel(
        _body,
        out_type=jax.ShapeDtypeStruct((B * S, D), jnp.float32),
        mesh=mesh,
        scratch_types=[
            pltpu.VMEM((SP_STRIDE, 128), jnp.int32),
            pltpu.VMEM((SQ_STRIDE, 128), jnp.int32),
            pltpu.VMEM((BPW * S, D), jnp.float32),
            pltpu.VMEM((SEQ_ROWS, D), jnp.float32),
            pltpu.SemaphoreType.DMA,
            pltpu.SemaphoreType.DMA,
        ],
        compiler_params=pltpu.CompilerParams(use_tc_tiling_on_sc=False),
    )
    out = run(idx27_2d, seq_2d, sp_flat, seq_table)
    return out.reshape(B, S, D)


# speedup vs baseline: 1.0809x; 1.0809x over previous
import functools

import jax
import jax.numpy as jnp
from jax import lax
from jax.experimental import pallas as pl
from jax.experimental.pallas import tpu as pltpu
from jax.experimental.pallas import tpu_sc as plsc

VOCAB = 100000
D = 16
F = 26
B = 4096
L = 50
S = F + 1
SEQ_BASE = F * VOCAB

_INFO = plsc.get_sparse_core_info()
NW = _INFO.num_cores * _INFO.num_subcores
BPW = B // NW
L_CHUNK = 25
INV_L = float(1.0 / L)


def _body(sidxT_hbm, sqidxT_hbm, tab_hbm, seq_hbm, out_hbm,
          idx_sp, idx_sq, out_blk, seq_rows, semS, semQ):
    wid = lax.axis_index("s") * _INFO.num_cores + lax.axis_index("c")
    col0 = wid * BPW

    pltpu.sync_copy(sidxT_hbm.at[:, pl.ds(col0, BPW)], idx_sp)
    pltpu.sync_copy(sqidxT_hbm.at[:, pl.ds(col0, BPW)], idx_sq)

    for f in range(F):
        for j in range(BPW // 16):
            sl = pl.ds(j * 16, 16)
            idx_sp[f, sl] = idx_sp[f, sl] + (f * VOCAB)
    sp_cps = []
    for f in range(F):
        sp_cps.append(pltpu.async_copy(
            tab_hbm.at[idx_sp.at[f]],
            out_blk.at[pl.ds(f * BPW, BPW)],
            semS))

    for c in range(2):
        sq_cps = []
        for i in range(L_CHUNK):
            l = c * L_CHUNK + i
            sq_cps.append(pltpu.async_copy(
                seq_hbm.at[idx_sq.at[l]],
                seq_rows.at[pl.ds(i * BPW, BPW)],
                semQ))
        for cp in sq_cps:
            cp.wait()

        def pool_one(bb, _, c=c):
            acc = seq_rows[bb, :]
            for i in range(1, L_CHUNK):
                acc = acc + seq_rows[i * BPW + bb, :]
            r = F * BPW + bb
            if c == 0:
                out_blk[r, :] = acc
            else:
                out_blk[r, :] = (out_blk[r, :] + acc) * INV_L
            return _

        lax.fori_loop(0, BPW, pool_one, None)

    for cp in sp_cps:
        cp.wait()
    pltpu.sync_copy(out_blk, out_hbm.at[pl.ds(wid * S * BPW, S * BPW)])


@functools.partial(jax.jit, static_argnames=())
def kernel(sparse_idx, seq_idx, sparse_tables, seq_table):
    sp_flat = sparse_tables.reshape(F * VOCAB, D)
    sidxT = sparse_idx.T
    sqidxT = seq_idx.T

    mesh = plsc.VectorSubcoreMesh(core_axis_name="c", subcore_axis_name="s")
    run = pl.kernel(
        _body,
        out_type=jax.ShapeDtypeStruct((NW * S * BPW, D), jnp.float32),
        mesh=mesh,
        scratch_types=[
            pltpu.VMEM((F, BPW), jnp.int32),
            pltpu.VMEM((L, BPW), jnp.int32),
            pltpu.VMEM((S * BPW, D), jnp.float32),
            pltpu.VMEM((L_CHUNK * BPW, D), jnp.float32),
            pltpu.SemaphoreType.DMA,
            pltpu.SemaphoreType.DMA,
        ],
        compiler_params=pltpu.CompilerParams(use_tc_tiling_on_sc=False),
    )
    out = run(sidxT, sqidxT, sp_flat, seq_table)
    return out.reshape(NW, S, BPW, D).transpose(0, 2, 1, 3).reshape(B, S, D)
